# baseline (speedup 1.0000x reference)
import functools

import jax
import jax.numpy as jnp
from jax import lax
from jax.experimental import pallas as pl
from jax.experimental.pallas import tpu as pltpu
from jax.experimental.pallas import tpu_sc as plsc

N_TAGS = 48
N_WORDS = 100000
B = 1024
L = 200

_NC = 2
_NS = 16
_NW = _NC * _NS
_LANES = 16

_sc_mesh = plsc.VectorSubcoreMesh(core_axis_name="c", subcore_axis_name="s")
_sc_params = pltpu.CompilerParams(use_tc_tiling_on_sc=False)
_sc_params_nlp = pltpu.CompilerParams(
    use_tc_tiling_on_sc=False, needs_layout_passes=False)

_WPW = 3136
N_WORDS_PAD = _NW * _WPW
_WB = 448
_NBLK_T = _WPW // _WB
_WG = _WB // _LANES


@functools.partial(
    pl.kernel,
    out_type=jax.ShapeDtypeStruct((N_WORDS_PAD, N_TAGS), jnp.float32),
    mesh=_sc_mesh,
    scratch_types=[
        pltpu.VMEM((N_TAGS, _WB), jnp.float32),
        pltpu.VMEM((N_TAGS, _WB), jnp.float32),
        pltpu.VMEM((_WB, N_TAGS), jnp.float32),
        pltpu.VMEM((_WB, N_TAGS), jnp.float32),
        pltpu.SemaphoreType.DMA,
        pltpu.SemaphoreType.DMA,
    ],
    compiler_params=_sc_params_nlp,
)
def _sc_transpose(emits_hbm, table_hbm, i0, i1, o0, o1, sem_in, sem_out):
    wid = lax.axis_index("s") * _NC + lax.axis_index("c")
    ivs = (i0, i1)
    ovs = (o0, o1)
    iota = lax.iota(jnp.int32, _LANES)

    def _w0(b):
        return jnp.minimum(wid * _WPW + b * _WB, N_WORDS - _WB)

    def fire_in(b):
        return pltpu.async_copy(
            emits_hbm.at[:, pl.ds(_w0(b), _WB)], ivs[b % 2], sem_in)

    def transpose(b):
        iv, ov = ivs[b % 2], ovs[b % 2]

        def body(wg, carry):
            rows = wg * _LANES + iota
            for t in range(N_TAGS):
                v = iv[t, pl.ds(wg * _LANES, _LANES)]
                plsc.store_scatter(
                    ov, [rows, jnp.full((_LANES,), t, jnp.int32)], v)
            return carry

        lax.fori_loop(0, _WG, body, 0)

    loads = [None] * _NBLK_T
    stores = [None] * _NBLK_T
    loads[0] = fire_in(0)
    for b in range(_NBLK_T):
        loads[b].wait()
        if b + 1 < _NBLK_T:
            loads[b + 1] = fire_in(b + 1)
        if b >= 2:
            stores[b - 2].wait()
        transpose(b)
        stores[b] = pltpu.async_copy(
            ovs[b % 2], table_hbm.at[pl.ds(_w0(b), _WB)], sem_out)
    stores[_NBLK_T - 2].wait()
    stores[_NBLK_T - 1].wait()


_BPW = B // _NW
_SG = _BPW // _LANES
_LPS = 4
_NSTR = L // _LPS
_RING = 5
_SPB = 5
_LBLK = _LPS * _SPB
_NLBLK = L // _LBLK
_GROWS = _LPS * _BPW


@functools.partial(
    pl.kernel,
    out_type=jax.ShapeDtypeStruct((L, N_TAGS, B), jnp.float32),
    mesh=_sc_mesh,
    scratch_types=[
        pltpu.VMEM((_NSTR, _GROWS), jnp.int32),
        pltpu.VMEM((_GROWS, N_TAGS), jnp.float32),
        pltpu.VMEM((_GROWS, N_TAGS), jnp.float32),
        pltpu.VMEM((_GROWS, N_TAGS), jnp.float32),
        pltpu.VMEM((_GROWS, N_TAGS), jnp.float32),
        pltpu.VMEM((_GROWS, N_TAGS), jnp.float32),
        pltpu.VMEM((_LBLK, N_TAGS, _BPW), jnp.float32),
        pltpu.VMEM((_LBLK, N_TAGS, _BPW), jnp.float32),
        pltpu.SemaphoreType.DMA,
        pltpu.SemaphoreType.DMA,
    ],
    compiler_params=_sc_params_nlp,
)
def _sc_gather(table_hbm, w2_hbm, out_hbm, idxv,
               r0, r1, r2, r3, r4, s0, s1, sem_g, sem_s):
    wid = lax.axis_index("s") * _NC + lax.axis_index("c")
    b0 = wid * _BPW
    rbufs = (r0, r1, r2, r3, r4)
    stg = (s0, s1)
    iota = lax.iota(jnp.int32, _LANES)
    rowsel = [[m * _BPW + sg * _LANES + iota for sg in range(_SG)]
              for m in range(_LPS)]

    pltpu.sync_copy(w2_hbm.at[wid], idxv)
    for u in range(_RING):
        pltpu.async_copy(table_hbm.at[idxv.at[u]], rbufs[u], sem_g)

    def wait_store(sub):
        pltpu.make_async_copy(
            stg[sub], out_hbm.at[pl.ds(0, _LBLK), :, pl.ds(b0, _BPW)],
            sem_s).wait()

    def do_block(blk, sub):
        stgb = stg[sub]
        for u in range(_SPB):
            s = blk * _SPB + u
            pltpu.make_async_copy(
                table_hbm.at[pl.ds(0, _GROWS)], rbufs[u], sem_g).wait()

            def body(j, carry, u=u):
                jv = jnp.full((_LANES,), 0, jnp.int32) + j
                for m in range(_LPS):
                    row = m * _BPW + j
                    lsel = jnp.full((_LANES,), u * _LPS + m, jnp.int32)
                    for t0 in range(N_TAGS // _LANES):
                        v = rbufs[u][row, pl.ds(t0 * _LANES, _LANES)]
                        plsc.store_scatter(
                            stgb, [lsel, t0 * _LANES + iota, jv], v)
                return carry

            lax.fori_loop(0, _BPW, body, 0)

            @pl.when(s + _RING < _NSTR)
            def _():
                pltpu.async_copy(
                    table_hbm.at[idxv.at[s + _RING]], rbufs[u], sem_g)
        pltpu.async_copy(
            stgb, out_hbm.at[pl.ds(blk * _LBLK, _LBLK), :, pl.ds(b0, _BPW)],
            sem_s)

    def outer(ob, carry):
        for sub in range(2):
            blk = ob * 2 + sub

            @pl.when(blk >= 2)
            def _():
                wait_store(sub)

            do_block(blk, sub)
        return carry

    lax.fori_loop(0, _NLBLK // 2, outer, 0)
    wait_store(0)
    wait_store(1)


def kernel(words, emits):
    table = _sc_transpose(emits)
    w2 = (words.reshape(_NW, _BPW, _NSTR, _LPS)
          .transpose(0, 2, 3, 1)
          .reshape(_NW, _NSTR, _GROWS))
    out_t = _sc_gather(table, w2)
    return jnp.transpose(out_t, (2, 0, 1))


# speedup vs baseline: 1.3095x; 1.3095x over previous
import functools

import jax
import jax.numpy as jnp
from jax import lax
from jax.experimental import pallas as pl
from jax.experimental.pallas import tpu as pltpu
from jax.experimental.pallas import tpu_sc as plsc

N_TAGS = 48
N_WORDS = 100000
B = 1024
L = 200
N_TOK = B * L

_NC = 2
_NS = 16
_NW = _NC * _NS
_LANES = 16

_sc_mesh = plsc.VectorSubcoreMesh(core_axis_name="c", subcore_axis_name="s")
_sc_params = pltpu.CompilerParams(use_tc_tiling_on_sc=False)
_sc_params_nlp = pltpu.CompilerParams(
    use_tc_tiling_on_sc=False, needs_layout_passes=False)

_WPW = 3136
N_WORDS_PAD = _NW * _WPW
_WB = 448
_NBLK_T = _WPW // _WB
_WG = _WB // _LANES


@functools.partial(
    pl.kernel,
    out_type=jax.ShapeDtypeStruct((N_WORDS_PAD, N_TAGS), jnp.float32),
    mesh=_sc_mesh,
    scratch_types=[
        pltpu.VMEM((N_TAGS, _WB), jnp.float32),
        pltpu.VMEM((N_TAGS, _WB), jnp.float32),
        pltpu.VMEM((_WB, N_TAGS), jnp.float32),
        pltpu.VMEM((_WB, N_TAGS), jnp.float32),
        pltpu.SemaphoreType.DMA,
        pltpu.SemaphoreType.DMA,
    ],
    compiler_params=_sc_params_nlp,
)
def _sc_transpose(emits_hbm, table_hbm, i0, i1, o0, o1, sem_in, sem_out):
    wid = lax.axis_index("s") * _NC + lax.axis_index("c")
    ivs = (i0, i1)
    ovs = (o0, o1)
    iota = lax.iota(jnp.int32, _LANES)

    def _w0(b):
        return jnp.minimum(wid * _WPW + b * _WB, N_WORDS - _WB)

    def fire_in(b):
        return pltpu.async_copy(
            emits_hbm.at[:, pl.ds(_w0(b), _WB)], ivs[b % 2], sem_in)

    def transpose(b):
        iv, ov = ivs[b % 2], ovs[b % 2]

        def body(wg, carry):
            rows = wg * _LANES + iota
            for t in range(N_TAGS):
                v = iv[t, pl.ds(wg * _LANES, _LANES)]
                plsc.store_scatter(
                    ov, [rows, jnp.full((_LANES,), t, jnp.int32)], v)
            return carry

        lax.fori_loop(0, _WG, body, 0)

    loads = [None] * _NBLK_T
    stores = [None] * _NBLK_T
    loads[0] = fire_in(0)
    for b in range(_NBLK_T):
        loads[b].wait()
        if b + 1 < _NBLK_T:
            loads[b + 1] = fire_in(b + 1)
        if b >= 2:
            stores[b - 2].wait()
        transpose(b)
        stores[b] = pltpu.async_copy(
            ovs[b % 2], table_hbm.at[pl.ds(_w0(b), _WB)], sem_out)
    stores[_NBLK_T - 2].wait()
    stores[_NBLK_T - 1].wait()


_SPW = B // _NW
_GSPLIT = (104, 96)
_BS = 4
_BTOK = _BS * L
_NBLK = _SPW // _BS


@functools.partial(
    pl.kernel,
    out_type=jax.ShapeDtypeStruct((N_TOK, N_TAGS), jnp.float32),
    mesh=_sc_mesh,
    scratch_types=[
        pltpu.VMEM((_SPW, L), jnp.int32),
        pltpu.VMEM((_BTOK, N_TAGS), jnp.float32),
        pltpu.VMEM((_BTOK, N_TAGS), jnp.float32),
        pltpu.VMEM((_BTOK, N_TAGS), jnp.float32),
        pltpu.SemaphoreType.DMA,
        pltpu.SemaphoreType.DMA,
    ],
    compiler_params=_sc_params,
)
def _sc_gather(table_hbm, words_hbm, out_hbm, idx_v, b0, b1, b2, sem_g, sem_s):
    wid = lax.axis_index("s") * _NC + lax.axis_index("c")
    sent0 = wid * _SPW
    bufs = (b0, b1, b2)
    pltpu.sync_copy(words_hbm.at[pl.ds(sent0, _SPW)], idx_v)

    def fire_block(g):
        buf = bufs[g % 3]
        cps = []
        for s in range(_BS):
            off = 0
            for sz in _GSPLIT:
                idx = idx_v.at[g * _BS + s, pl.ds(off, sz)]
                dst = buf.at[pl.ds(s * L + off, sz)]
                cps.append(pltpu.async_copy(table_hbm.at[idx], dst, sem_g))
                off += sz
        return cps

    gathers = [None] * _NBLK
    stores = [None] * _NBLK
    gathers[0] = fire_block(0)
    gathers[1] = fire_block(1)
    for g in range(_NBLK):
        for cp in gathers[g]:
            cp.wait()
        stores[g] = pltpu.async_copy(
            bufs[g % 3],
            out_hbm.at[pl.ds((sent0 + g * _BS) * L, _BTOK)], sem_s)
        if g + 2 < _NBLK:
            if g >= 1:
                stores[g - 1].wait()
            gathers[g + 2] = fire_block(g + 2)
    stores[_NBLK - 2].wait()
    stores[_NBLK - 1].wait()


def kernel(words, emits):
    table = _sc_transpose(emits)
    out = _sc_gather(table, words)
    return out.reshape(B, L, N_TAGS)
